# baseline (speedup 1.0000x reference)
import jax
import jax.numpy as jnp
from jax.experimental import pallas as pl

_B, _N, _C = 64, 128, 3
_HID = 64
_NUM_LAYERS = 3
_N_CONVS = _NUM_LAYERS * 2 + 1
_R = _N // 2
_W = 2 * _HID


def _blockdiag(w):
    z = jnp.zeros((_HID, _HID), jnp.float32)
    top = jnp.concatenate([w, z], axis=1)
    bot = jnp.concatenate([z, w], axis=1)
    return jnp.concatenate([top, bot], axis=0)


def _stack22(w):
    t = jnp.concatenate([w, w], axis=1)
    return jnp.concatenate([t, t], axis=0)


def _fused_kernel(kp_ref, win_ref, bin_ref, wr_ref, wn_ref, bc_ref,
                  wh_ref, bh_ref, out_ref):
    kp = kp_ref[...]
    li = jax.lax.broadcasted_iota(jnp.int32, (_N * _C, _R), 0)
    pi = jax.lax.broadcasted_iota(jnp.int32, (_N * _C, _R), 1)
    win2 = win_ref[...]
    zc = jnp.zeros((1, _HID), jnp.float32)
    bin2 = jnp.concatenate([bin_ref[...], bin_ref[...]], axis=1)
    xp3 = jnp.broadcast_to(bin2.reshape(1, 1, _W), (_R, _B, _W))
    for s in range(2):
        for c in range(_C):
            sel = (li == 6 * pi + 3 * s + c).astype(jnp.float32)
            kpsc = jnp.dot(kp, sel, preferred_element_type=jnp.float32)
            kpscT = kpsc.T
            wrow = win2[c:c + 1, :]
            if s == 0:
                wlane = jnp.concatenate([wrow, zc], axis=1)
            else:
                wlane = jnp.concatenate([zc, wrow], axis=1)
            xp3 = xp3 + kpscT[:, :, None] * wlane.reshape(1, 1, _W)
    x = xp3.reshape(_R * _B, _W)

    def sage(x, i):
        s2 = jnp.sum(x.reshape(_R, _B, _W), axis=0)
        b2 = jnp.concatenate([bc_ref[i], bc_ref[i]], axis=0)
        agg = jnp.dot(s2 * (1.0 / _N), _stack22(wn_ref[i]),
                      preferred_element_type=jnp.float32) + b2[None, :]
        root = jnp.dot(x, _blockdiag(wr_ref[i]),
                       preferred_element_type=jnp.float32)
        aggb = jnp.broadcast_to(agg[None, :, :], (_R, _B, _W))
        return root + aggb.reshape(_R * _B, _W)

    for l in range(_NUM_LAYERS):
        h = jnp.maximum(sage(x, 2 * l), 0.0)
        h = sage(h, 2 * l + 1)
        x = jnp.maximum(h + x, 0.0)
    x = sage(x, _N_CONVS - 1)

    t = x.reshape(_R, _B, _W) * wh_ref[...][:, None, :]
    g = jnp.sum(jnp.sum(t, axis=0), axis=1)
    out_ref[...] = g[:, None] + bh_ref[...][None, :]


def kernel(keypoints, W_in, b_in, W_root, W_neigh, b_conv, W_head, b_head):
    kp384 = keypoints.reshape(_B, _N * _C)
    return pl.pallas_call(
        _fused_kernel,
        out_shape=jax.ShapeDtypeStruct((_B, 1), jnp.float32),
    )(kp384, W_in, b_in.reshape(1, _HID), W_root, W_neigh, b_conv,
      W_head.reshape(_R, _W), b_head)


# speedup vs baseline: 1.2425x; 1.2425x over previous
import jax
import jax.numpy as jnp
from jax.experimental import pallas as pl

_B, _N, _C = 64, 128, 3
_HID = 64
_NUM_LAYERS = 3
_N_CONVS = _NUM_LAYERS * 2 + 1
_R = _N // 2
_W = 2 * _HID


def _blockdiag(w):
    z = jnp.zeros((_HID, _HID), jnp.float32)
    top = jnp.concatenate([w, z], axis=1)
    bot = jnp.concatenate([z, w], axis=1)
    return jnp.concatenate([top, bot], axis=0)


def _stack22(w):
    t = jnp.concatenate([w, w], axis=1)
    return jnp.concatenate([t, t], axis=0)


def _fused_kernel(kp_ref, win_ref, bin_ref, wr_ref, wn_ref, bc_ref,
                  wh_ref, bh_ref, out_ref):
    kpT = kp_ref[...].T.reshape(_R, 2 * _C, _B)
    kpe6 = jnp.swapaxes(kpT, 1, 2)
    ones = jnp.ones((_R, _B, 1), jnp.float32)
    kpe7 = jnp.concatenate([kpe6, ones], axis=2).reshape(_R * _B, 2 * _C + 1)
    win = win_ref[...]
    zc = jnp.zeros((_C, _HID), jnp.float32)
    bin2 = jnp.concatenate([bin_ref[...], bin_ref[...]], axis=1)
    win_blk = jnp.concatenate(
        [jnp.concatenate([win, zc], axis=1),
         jnp.concatenate([zc, win], axis=1),
         bin2], axis=0)
    x = jnp.dot(kpe7, win_blk, preferred_element_type=jnp.float32)

    def sage(x, i):
        s2 = jnp.sum(x.reshape(_R, _B, _W), axis=0)
        b2 = jnp.concatenate([bc_ref[i], bc_ref[i]], axis=0)
        agg = jnp.dot(s2 * (1.0 / _N), _stack22(wn_ref[i]),
                      preferred_element_type=jnp.float32) + b2[None, :]
        root = jnp.dot(x, _blockdiag(wr_ref[i]),
                       preferred_element_type=jnp.float32)
        aggb = jnp.broadcast_to(agg[None, :, :], (_R, _B, _W))
        return root + aggb.reshape(_R * _B, _W)

    for l in range(_NUM_LAYERS):
        h = jnp.maximum(sage(x, 2 * l), 0.0)
        h = sage(h, 2 * l + 1)
        x = jnp.maximum(h + x, 0.0)
    x = sage(x, _N_CONVS - 1)

    t = x.reshape(_R, _B, _W) * wh_ref[...][:, None, :]
    g = jnp.sum(jnp.sum(t, axis=0), axis=1)
    out_ref[...] = g[:, None] + bh_ref[...][None, :]


def kernel(keypoints, W_in, b_in, W_root, W_neigh, b_conv, W_head, b_head):
    kp384 = keypoints.reshape(_B, _N * _C)
    return pl.pallas_call(
        _fused_kernel,
        out_shape=jax.ShapeDtypeStruct((_B, 1), jnp.float32),
    )(kp384, W_in, b_in.reshape(1, _HID), W_root, W_neigh, b_conv,
      W_head.reshape(_R, _W), b_head)
